# baseline (speedup 1.0000x reference)
import functools

import jax
import jax.numpy as jnp
from jax import lax
from jax.experimental import pallas as pl
from jax.experimental.pallas import tpu as pltpu
from jax.experimental.pallas import tpu_sc as plsc

NC = 2
NS = 16
NW = NC * NS

LINE = 128
VREG = 16
UNROLL = 8


def _gather_kernel(S0, S1, D, table_hbm, idx_hbm, out_hbm,
                   idx_v, rows0, rows1, comp0, comp1,
                   gsem0, gsem1, wsem0, wsem1):
    b_per_w = (S0 * S1) // NW
    rows_per_w = b_per_w // S1
    chunks = [min(128, S1 - o) for o in range(0, S1, 128)]
    wid = lax.axis_index("s") * NC + lax.axis_index("c")
    tok_base = wid * b_per_w
    row_base = wid * rows_per_w
    pltpu.sync_copy(idx_hbm.at[pl.ds(tok_base, b_per_w)], idx_v)

    def fire_gather(g, rows_v, gsem):
        cps = []
        off = 0
        for c in chunks:
            cps.append(pltpu.async_copy(
                table_hbm.at[idx_v.at[pl.ds(g * S1 + off, c)]],
                rows_v.at[pl.ds(off, c), :],
                gsem,
            ))
            off += c
        return cps

    def compact(rows_v, comp_v):
        @plsc.parallel_loop(0, S1, unroll=UNROLL)
        def _(r):
            for c0 in range(0, D, VREG):
                comp_v[r, pl.ds(c0, VREG)] = rows_v[r, pl.ds(c0, VREG)]

    @pl.loop(0, rows_per_w)
    def _(g):
        ga = fire_gather(g, rows0, gsem0)
        for c in ga:
            c.wait()
        compact(rows0, comp0)
        pltpu.async_copy(comp0, out_hbm.at[row_base + g], wsem0).wait()


def kernel(phonemes, table):
    S0, S1 = phonemes.shape
    B = S0 * S1
    V, D = table.shape
    idx = phonemes.reshape(B).astype(jnp.int32)
    table_pad = jnp.pad(table, ((0, 0), (0, LINE - D)))

    mesh = plsc.VectorSubcoreMesh(core_axis_name="c", subcore_axis_name="s")
    b_per_w = B // NW

    k = pl.kernel(
        functools.partial(_gather_kernel, S0, S1, D),
        out_type=jax.ShapeDtypeStruct((S0, S1, D), table.dtype),
        mesh=mesh,
        scratch_types=[
            pltpu.VMEM((b_per_w,), jnp.int32),
            pltpu.VMEM((S1, LINE), jnp.float32),
            pltpu.VMEM((S1, LINE), jnp.float32),
            pltpu.VMEM((S1, D), jnp.float32),
            pltpu.VMEM((S1, D), jnp.float32),
            pltpu.SemaphoreType.DMA,
            pltpu.SemaphoreType.DMA,
            pltpu.SemaphoreType.DMA,
            pltpu.SemaphoreType.DMA,
        ],
    )
    return k(table_pad, idx)


# speedup vs baseline: 1.0003x; 1.0003x over previous
import functools

import jax
import jax.numpy as jnp
from jax import lax
from jax.experimental import pallas as pl
from jax.experimental.pallas import tpu as pltpu
from jax.experimental.pallas import tpu_sc as plsc

NC = 2
NS = 16
NW = NC * NS

LINE = 128
VREG = 16
UNROLL = 8


def _gather_kernel(S0, S1, D, table_hbm, idx_hbm, out_hbm,
                   idx_v, rows0, rows1, comp0, comp1,
                   gsem0, gsem1, wsem0, wsem1):
    b_per_w = (S0 * S1) // NW
    rows_per_w = b_per_w // S1
    chunks = [min(128, S1 - o) for o in range(0, S1, 128)]
    wid = lax.axis_index("s") * NC + lax.axis_index("c")
    tok_base = wid * b_per_w
    row_base = wid * rows_per_w
    pltpu.sync_copy(idx_hbm.at[pl.ds(tok_base, b_per_w)], idx_v)

    def fire_gather(g, rows_v, gsem):
        cps = []
        off = 0
        for c in chunks:
            cps.append(pltpu.async_copy(
                table_hbm.at[idx_v.at[pl.ds(g * S1 + off, c)]],
                rows_v.at[pl.ds(off, c), :],
                gsem,
            ))
            off += c
        return cps

    def compact(rows_v, comp_v):
        @plsc.parallel_loop(0, S1, unroll=UNROLL)
        def _(r):
            for c0 in range(0, D, VREG):
                comp_v[0, r, pl.ds(c0, VREG)] = rows_v[r, pl.ds(c0, VREG)]

    @pl.loop(0, rows_per_w)
    def _(g):
        ga = fire_gather(g, rows0, gsem0)
        for c in ga:
            c.wait()
        compact(rows0, comp0)
        pltpu.async_copy(comp0, out_hbm.at[pl.ds(row_base + g, 1)],
                         wsem0).wait()


def kernel(phonemes, table):
    S0, S1 = phonemes.shape
    B = S0 * S1
    V, D = table.shape
    idx = phonemes.reshape(B).astype(jnp.int32)
    table_pad = jnp.pad(table, ((0, 0), (0, LINE - D)))

    mesh = plsc.VectorSubcoreMesh(core_axis_name="c", subcore_axis_name="s")
    b_per_w = B // NW

    k = pl.kernel(
        functools.partial(_gather_kernel, S0, S1, D),
        out_type=jax.ShapeDtypeStruct((S0, S1, D), table.dtype),
        mesh=mesh,
        scratch_types=[
            pltpu.VMEM((b_per_w,), jnp.int32),
            pltpu.VMEM((S1, LINE), jnp.float32),
            pltpu.VMEM((S1, LINE), jnp.float32),
            pltpu.VMEM((1, S1, D), jnp.float32),
            pltpu.VMEM((1, S1, D), jnp.float32),
            pltpu.SemaphoreType.DMA,
            pltpu.SemaphoreType.DMA,
            pltpu.SemaphoreType.DMA,
            pltpu.SemaphoreType.DMA,
        ],
    )
    return k(table_pad, idx)


# speedup vs baseline: 1.3636x; 1.3632x over previous
import functools

import jax
import jax.numpy as jnp
from jax import lax
from jax.experimental import pallas as pl
from jax.experimental.pallas import tpu as pltpu
from jax.experimental.pallas import tpu_sc as plsc

NC = 2
NS = 16
NW = NC * NS

LINE = 128
G = 80
DEPTH = 4
VREG = 16
UNROLL = 8


def _gather_kernel(B, D, table_hbm, idx_hbm, out_hbm, idx_v, *bufs):
    rows = bufs[0:DEPTH]
    comp = bufs[DEPTH:2 * DEPTH]
    gsem = bufs[2 * DEPTH:3 * DEPTH]
    wsem = bufs[3 * DEPTH:4 * DEPTH]

    b_per_w = B // NW
    groups = b_per_w // G
    wid = lax.axis_index("s") * NC + lax.axis_index("c")
    tok_base = wid * b_per_w
    pltpu.sync_copy(idx_hbm.at[pl.ds(tok_base, b_per_w)], idx_v)

    def compact(rows_v, comp_v):
        @plsc.parallel_loop(0, G, unroll=UNROLL)
        def _(r):
            for c0 in range(0, D, VREG):
                comp_v[r, pl.ds(c0, VREG)] = rows_v[r, pl.ds(c0, VREG)]

    @pl.loop(0, groups // DEPTH)
    def _(q):
        g0 = q * DEPTH
        ga = [pltpu.async_copy(
                  table_hbm.at[idx_v.at[pl.ds((g0 + i) * G, G)]],
                  rows[i], gsem[i])
              for i in range(DEPTH)]
        ws = []
        for i in range(DEPTH):
            ga[i].wait()
            compact(rows[i], comp[i])
            ws.append(pltpu.async_copy(
                comp[i],
                out_hbm.at[pl.ds(tok_base + (g0 + i) * G, G)],
                wsem[i]))
        for w in ws:
            w.wait()


def kernel(phonemes, table):
    S0, S1 = phonemes.shape
    B = S0 * S1
    V, D = table.shape
    idx = phonemes.reshape(B).astype(jnp.int32)
    table_pad = jnp.pad(table, ((0, 0), (0, LINE - D)))

    mesh = plsc.VectorSubcoreMesh(core_axis_name="c", subcore_axis_name="s")
    b_per_w = B // NW

    k = pl.kernel(
        functools.partial(_gather_kernel, B, D),
        out_type=jax.ShapeDtypeStruct((B, D), table.dtype),
        mesh=mesh,
        scratch_types=(
            [pltpu.VMEM((b_per_w,), jnp.int32)]
            + [pltpu.VMEM((G, LINE), jnp.float32)] * DEPTH
            + [pltpu.VMEM((G, D), jnp.float32)] * DEPTH
            + [pltpu.SemaphoreType.DMA] * (2 * DEPTH)
        ),
    )
    return k(table_pad, idx).reshape(S0, S1, D)
